# baseline (speedup 1.0000x reference)
import functools

import jax
import jax.numpy as jnp
import numpy as np
from jax import lax
from jax.experimental import pallas as pl
from jax.experimental.pallas import tpu as pltpu
from jax.experimental.pallas import tpu_sc as plsc

BB = 32
GG = 10000
SS = 16
LL = 4
SL = SS * LL
GAMMA = 0.001
INV_GAMMA = 1.0 / GAMMA

NC, NS = 2, 16
NW = NC * NS
GPW = 320
GPAD = NW * GPW
NGB = GPW // 16
NCH = BB // 4

NTAB = 17536


def _softplus_table() -> np.ndarray:
    bits = (np.arange(NTAB, dtype=np.uint32) << 16).view(np.float32)
    return np.log1p(np.exp(-bits.astype(np.float64))).astype(np.float32)


_TAB = _softplus_table()


def _hi(w):
    return lax.bitcast_convert_type(w & jnp.int32(-65536), jnp.float32)


def _lo(w):
    return lax.bitcast_convert_type(w << 16, jnp.float32)


def _gtab(tab_v, d):
    bits = lax.bitcast_convert_type(d, jnp.int32)
    return plsc.load_gather(tab_v, [lax.shift_right_logical(bits, 16)])


def _sa(tab_v, a, b):
    return jnp.minimum(a, b) - _gtab(tab_v, jnp.abs(a - b))


def _so(tab_v, a, b):
    return jnp.maximum(a, b) + _gtab(tab_v, jnp.abs(a - b))


def _make_sc_call(interpret=False):
    mesh = plsc.VectorSubcoreMesh(
        core_axis_name="c", subcore_axis_name="s",
        num_cores=NC, num_subcores=NS)

    @functools.partial(
        pl.kernel,
        interpret=interpret,
        out_type=jax.ShapeDtypeStruct((BB * GG,), jnp.float32),
        mesh=mesh,
        compiler_params=pltpu.CompilerParams(needs_layout_passes=False),
        scratch_types=[
            pltpu.VMEM((SL * GPW,), jnp.int32),
            pltpu.VMEM((SL * GPW,), jnp.int32),
            pltpu.VMEM((NTAB,), jnp.float32),
            pltpu.VMEM((GG,), jnp.int32),
            pltpu.VMEM((GG,), jnp.int32),
            pltpu.VMEM((GG,), jnp.int32),
            pltpu.VMEM((GG,), jnp.int32),
            pltpu.VMEM((GPW,), jnp.float32),
            pltpu.VMEM((GPW,), jnp.float32),
            pltpu.VMEM((GPW,), jnp.float32),
            pltpu.VMEM((GPW,), jnp.float32),
            pltpu.SemaphoreType.DMA,
            pltpu.SemaphoreType.DMA,
        ],
    )
    def sc_clause(xp_hbm, idx_hbm, tab_hbm, out_hbm, raw_v, idx_v, tab_v,
                  xa0_v, xa1_v, xb0_v, xb1_v, o0_v, o1_v, o2_v, o3_v,
                  sem_a, sem_b):
        wid = lax.axis_index("s") * NC + lax.axis_index("c")
        gbase = jnp.minimum(wid * GPW, GG - GPW)
        pltpu.sync_copy(idx_hbm.at[pl.ds(gbase * SL, SL * GPW)], raw_v)

        def xcopy(ch, x0_v, x1_v, sem):
            return (pltpu.make_async_copy(
                        xp_hbm.at[pl.ds((2 * ch) * GG, GG)], x0_v, sem),
                    pltpu.make_async_copy(
                        xp_hbm.at[pl.ds((2 * ch + 1) * GG, GG)], x1_v, sem))

        for cp in xcopy(0, xa0_v, xa1_v, sem_a):
            cp.start()
        pltpu.sync_copy(tab_hbm, tab_v)

        i64v = lax.iota(jnp.int32, 16) * SL

        def tr_body(cb, carry):
            base = cb * (16 * SL)
            for j in range(SL):
                v = plsc.load_gather(raw_v, [i64v + (base + j)])
                idx_v[pl.ds(j * GPW + cb * 16, 16)] = v
            return carry

        lax.fori_loop(0, NGB, tr_body, 0)
        orefs = (o0_v, o1_v, o2_v, o3_v)

        def compute_chunk(ch, x0_v, x1_v):
            def gb_body(gb, inner):
                col = gb * 16
                stacks = [[] for _ in range(4)]
                for s in range(SS):
                    ws = []
                    for l in range(LL):
                        iv = idx_v[pl.ds((s * LL + l) * GPW + col, 16)]
                        ws.append(plsc.load_gather(x0_v, [iv]))
                        ws.append(plsc.load_gather(x1_v, [iv]))
                    sa1 = [[None, None], [None, None]]
                    for src in range(2):
                        aa = [plsc.bitcast(ws[2 * l + src], jnp.bfloat16)
                              for l in range(LL)]
                        for pair in range(2):
                            x1, x2 = aa[2 * pair], aa[2 * pair + 1]
                            m = plsc.bitcast(jnp.minimum(x1, x2), jnp.int32)
                            dw = plsc.bitcast(jnp.abs(x1 - x2), jnp.int32)
                            gh = plsc.load_gather(
                                tab_v, [lax.shift_right_logical(dw, 16)])
                            gl = plsc.load_gather(
                                tab_v, [dw & jnp.int32(0xFFFF)])
                            sa1[src][pair] = (_hi(m) - gh, _lo(m) - gl)
                    for k in range(4):
                        off = k >> 1
                        half = k & 1
                        p = _sa(tab_v, sa1[off][0][half], sa1[off][1][half])
                        cnt = s + 1
                        while cnt % 2 == 0:
                            p = _so(tab_v, stacks[k].pop(), p)
                            cnt //= 2
                        stacks[k].append(p)
                for k in range(4):
                    orefs[k][pl.ds(col, 16)] = stacks[k][0] * GAMMA
                return inner

            lax.fori_loop(0, NGB, gb_body, 0)
            for k, brow in enumerate((2 * ch, 2 * ch + 16,
                                      2 * ch + 1, 2 * ch + 17)):
                pltpu.sync_copy(
                    orefs[k], out_hbm.at[pl.ds(brow * GG + gbase, GPW)])

        def dc_body(i, carry):
            ch_a = 2 * i
            ch_b = 2 * i + 1
            for cp in xcopy(ch_a, xa0_v, xa1_v, sem_a):
                cp.wait()
            for cp in xcopy(ch_b, xb0_v, xb1_v, sem_b):
                cp.start()
            compute_chunk(ch_a, xa0_v, xa1_v)
            for cp in xcopy(ch_b, xb0_v, xb1_v, sem_b):
                cp.wait()

            @pl.when(i < (NCH // 2) - 1)
            def _():
                for cp in xcopy(ch_a + 2, xa0_v, xa1_v, sem_a):
                    cp.start()

            compute_chunk(ch_b, xb0_v, xb1_v)
            return carry

        lax.fori_loop(0, NCH // 2, dc_body, 0)

    return sc_clause


_SC_CALL_CACHE = []


def kernel(x, I_i):
    if not _SC_CALL_CACHE:
        _SC_CALL_CACHE.append(_make_sc_call())
    sc_clause = _SC_CALL_CACHE[0]
    y16 = (x * INV_GAMMA).astype(jnp.bfloat16)
    hi = lax.bitcast_convert_type(y16[:16], jnp.uint16).astype(jnp.uint32) << 16
    lo = lax.bitcast_convert_type(y16[16:], jnp.uint16).astype(jnp.uint32)
    xp = lax.bitcast_convert_type(hi | lo, jnp.int32)
    idx = I_i.reshape(-1).astype(jnp.int32)
    out = sc_clause(xp.reshape(-1), idx, jnp.asarray(_TAB))
    return out.reshape(BB, GG)


# speedup vs baseline: 2.0837x; 2.0837x over previous
import functools

import jax
import jax.numpy as jnp
import numpy as np
from jax import lax
from jax.experimental import pallas as pl
from jax.experimental.pallas import tpu as pltpu
from jax.experimental.pallas import tpu_sc as plsc

BB = 32
GG = 10000
SS = 16
LL = 4
SL = SS * LL
GAMMA = 0.001
INV_GAMMA = 1.0 / GAMMA

NC, NS = 2, 16
NW = NC * NS
GPW = 320
GPAD = NW * GPW
NGB = GPW // 16
NCH = BB // 4

NTAB = 17536


def _softplus_table() -> np.ndarray:
    bits = (np.arange(NTAB, dtype=np.uint32) << 16).view(np.float32)
    return np.log1p(np.exp(-bits.astype(np.float64))).astype(np.float32)


_TAB = _softplus_table()


def _hi(w):
    return lax.bitcast_convert_type(w & jnp.int32(-65536), jnp.float32)


def _lo(w):
    return lax.bitcast_convert_type(w << 16, jnp.float32)


def _gtab(tab_v, d):
    bits = lax.bitcast_convert_type(d, jnp.int32)
    return plsc.load_gather(tab_v, [lax.shift_right_logical(bits, 16)])


def _sa(tab_v, a, b):
    return jnp.minimum(a, b) - _gtab(tab_v, jnp.abs(a - b))


def _so(tab_v, a, b):
    return jnp.maximum(a, b) + _gtab(tab_v, jnp.abs(a - b))


def _make_sc_call(interpret=False):
    mesh = plsc.VectorSubcoreMesh(
        core_axis_name="c", subcore_axis_name="s",
        num_cores=NC, num_subcores=NS)

    @functools.partial(
        pl.kernel,
        interpret=interpret,
        out_type=jax.ShapeDtypeStruct((BB * GPAD,), jnp.float32),
        mesh=mesh,
        compiler_params=pltpu.CompilerParams(needs_layout_passes=False),
        scratch_types=[
            pltpu.VMEM((SL * GPW // 2,), jnp.int32),
            pltpu.VMEM((NTAB,), jnp.float32),
            pltpu.VMEM((GG,), jnp.int32),
            pltpu.VMEM((GG,), jnp.int32),
            pltpu.VMEM((GG,), jnp.int32),
            pltpu.VMEM((GG,), jnp.int32),
            pltpu.VMEM((GPW,), jnp.float32),
            pltpu.VMEM((GPW,), jnp.float32),
            pltpu.VMEM((GPW,), jnp.float32),
            pltpu.VMEM((GPW,), jnp.float32),
            pltpu.SemaphoreType.DMA,
            pltpu.SemaphoreType.DMA,
        ],
    )
    def sc_clause(xp_hbm, idx_hbm, tab_hbm, out_hbm, idx_v, tab_v,
                  xa0_v, xa1_v, xb0_v, xb1_v, o0_v, o1_v, o2_v, o3_v,
                  sem_a, sem_b):
        wid = lax.axis_index("s") * NC + lax.axis_index("c")
        nwords = SL * GPW // 2
        pltpu.sync_copy(idx_hbm.at[pl.ds(wid * nwords, nwords)], idx_v)

        def xcopy(ch, x0_v, x1_v, sem):
            return (pltpu.make_async_copy(
                        xp_hbm.at[pl.ds((2 * ch) * GG, GG)], x0_v, sem),
                    pltpu.make_async_copy(
                        xp_hbm.at[pl.ds((2 * ch + 1) * GG, GG)], x1_v, sem))

        for cp in xcopy(0, xa0_v, xa1_v, sem_a):
            cp.start()
        pltpu.sync_copy(tab_hbm, tab_v)
        orefs = (o0_v, o1_v, o2_v, o3_v)

        def compute_chunk(ch, x0_v, x1_v):
            def gb_body(gb, inner):
                col = gb * 16
                stacks = [[] for _ in range(4)]
                for s in range(SS):
                    ws = []
                    for l2 in range(2):
                        w = idx_v[pl.ds((s * 2 + l2) * GPW + col, 16)]
                        for iv in (w & jnp.int32(0xFFFF),
                                   lax.shift_right_logical(w, 16)):
                            ws.append(plsc.load_gather(x0_v, [iv]))
                            ws.append(plsc.load_gather(x1_v, [iv]))
                    sa1 = [[None, None], [None, None]]
                    for src in range(2):
                        aa = [plsc.bitcast(ws[2 * l + src], jnp.bfloat16)
                              for l in range(LL)]
                        for pair in range(2):
                            x1, x2 = aa[2 * pair], aa[2 * pair + 1]
                            m = plsc.bitcast(jnp.minimum(x1, x2), jnp.int32)
                            dw = plsc.bitcast(jnp.abs(x1 - x2), jnp.int32)
                            gh = plsc.load_gather(
                                tab_v, [lax.shift_right_logical(dw, 16)])
                            gl = plsc.load_gather(
                                tab_v, [dw & jnp.int32(0xFFFF)])
                            sa1[src][pair] = (_hi(m) - gh, _lo(m) - gl)
                    for k in range(4):
                        off = k >> 1
                        half = k & 1
                        p = _sa(tab_v, sa1[off][0][half], sa1[off][1][half])
                        cnt = s + 1
                        while cnt % 2 == 0:
                            p = _so(tab_v, stacks[k].pop(), p)
                            cnt //= 2
                        stacks[k].append(p)
                for k in range(4):
                    orefs[k][pl.ds(col, 16)] = stacks[k][0] * GAMMA
                return inner

            lax.fori_loop(0, NGB, gb_body, 0)
            for k, brow in enumerate((2 * ch, 2 * ch + 16,
                                      2 * ch + 1, 2 * ch + 17)):
                pltpu.sync_copy(
                    orefs[k], out_hbm.at[pl.ds(brow * GPAD + wid * GPW, GPW)])

        def dc_body(i, carry):
            ch_a = 2 * i
            ch_b = 2 * i + 1
            for cp in xcopy(ch_a, xa0_v, xa1_v, sem_a):
                cp.wait()
            for cp in xcopy(ch_b, xb0_v, xb1_v, sem_b):
                cp.start()
            compute_chunk(ch_a, xa0_v, xa1_v)
            for cp in xcopy(ch_b, xb0_v, xb1_v, sem_b):
                cp.wait()

            @pl.when(i < (NCH // 2) - 1)
            def _():
                for cp in xcopy(ch_a + 2, xa0_v, xa1_v, sem_a):
                    cp.start()

            compute_chunk(ch_b, xb0_v, xb1_v)
            return carry

        lax.fori_loop(0, NCH // 2, dc_body, 0)

    return sc_clause


_SC_CALL_CACHE = []


def kernel(x, I_i):
    if not _SC_CALL_CACHE:
        _SC_CALL_CACHE.append(_make_sc_call())
    sc_clause = _SC_CALL_CACHE[0]
    y16 = (x * INV_GAMMA).astype(jnp.bfloat16)
    hi = lax.bitcast_convert_type(y16[:16], jnp.uint16).astype(jnp.uint32) << 16
    lo = lax.bitcast_convert_type(y16[16:], jnp.uint16).astype(jnp.uint32)
    xp = lax.bitcast_convert_type(hi | lo, jnp.int32)
    idx = I_i.reshape(GG, SL).astype(jnp.int32)
    idx = jnp.pad(idx, ((0, GPAD - GG), (0, 0)))
    idx = idx.reshape(NW, GPW, SL).transpose(0, 2, 1)
    idx = idx[:, 0::2, :] | (idx[:, 1::2, :] << 16)
    out = sc_clause(xp.reshape(-1), idx.reshape(-1), jnp.asarray(_TAB))
    return out.reshape(BB, GPAD)[:, :GG]


# speedup vs baseline: 2.1250x; 1.0198x over previous
import functools

import jax
import jax.numpy as jnp
import numpy as np
from jax import lax
from jax.experimental import pallas as pl
from jax.experimental.pallas import tpu as pltpu
from jax.experimental.pallas import tpu_sc as plsc

BB = 32
GG = 10000
SS = 16
LL = 4
SL = SS * LL
GAMMA = 0.001
INV_GAMMA = 1.0 / GAMMA

NC, NS = 2, 16
NW = NC * NS
GPW = 320
GPAD = NW * GPW
NGB = GPW // 16
NCH = BB // 4

NTAB = 17536


def _softplus_table() -> np.ndarray:
    bits = (np.arange(NTAB, dtype=np.uint32) << 16).view(np.float32)
    return np.log1p(np.exp(-bits.astype(np.float64))).astype(np.float32)


_TAB = _softplus_table()


def _hi(w):
    return lax.bitcast_convert_type(w & jnp.int32(-65536), jnp.float32)


def _lo(w):
    return lax.bitcast_convert_type(w << 16, jnp.float32)


def _gtab(tab_v, d):
    bits = lax.bitcast_convert_type(d, jnp.int32)
    return plsc.load_gather(tab_v, [lax.shift_right_logical(bits, 16)])


def _sa(tab_v, a, b):
    return jnp.minimum(a, b) - _gtab(tab_v, jnp.abs(a - b))


def _so(tab_v, a, b):
    return jnp.maximum(a, b) + _gtab(tab_v, jnp.abs(a - b))


def _make_sc_call(interpret=False):
    mesh = plsc.VectorSubcoreMesh(
        core_axis_name="c", subcore_axis_name="s",
        num_cores=NC, num_subcores=NS)

    @functools.partial(
        pl.kernel,
        interpret=interpret,
        out_type=jax.ShapeDtypeStruct((BB * GPAD,), jnp.float32),
        mesh=mesh,
        compiler_params=pltpu.CompilerParams(needs_layout_passes=False),
        scratch_types=[
            pltpu.VMEM((SL * GPW // 2,), jnp.int32),
            pltpu.VMEM((NTAB,), jnp.float32),
            pltpu.VMEM((GG,), jnp.int32),
            pltpu.VMEM((GG,), jnp.int32),
            pltpu.VMEM((GG,), jnp.int32),
            pltpu.VMEM((GG,), jnp.int32),
            [pltpu.VMEM((GPW,), jnp.float32) for _ in range(8)],
            pltpu.SemaphoreType.DMA,
            pltpu.SemaphoreType.DMA,
            pltpu.SemaphoreType.DMA,
        ],
    )
    def sc_clause(xp_hbm, idx_hbm, tab_hbm, out_hbm, idx_v, tab_v,
                  xa0_v, xa1_v, xb0_v, xb1_v, orows,
                  sem_a, sem_b, sem_o):
        wid = lax.axis_index("s") * NC + lax.axis_index("c")
        nwords = SL * GPW // 2
        pltpu.sync_copy(idx_hbm.at[pl.ds(wid * nwords, nwords)], idx_v)

        def xcopy(ch, x0_v, x1_v, sem):
            return (pltpu.make_async_copy(
                        xp_hbm.at[pl.ds((2 * ch) * GG, GG)], x0_v, sem),
                    pltpu.make_async_copy(
                        xp_hbm.at[pl.ds((2 * ch + 1) * GG, GG)], x1_v, sem))

        for cp in xcopy(0, xa0_v, xa1_v, sem_a):
            cp.start()
        pltpu.sync_copy(tab_hbm, tab_v)

        def compute_chunk(ch, x0_v, x1_v, orefs):
            def gb_body(gb, inner):
                col = gb * 16
                stacks = [[] for _ in range(4)]
                for s in range(SS):
                    ws = []
                    for l2 in range(2):
                        w = idx_v[pl.ds((s * 2 + l2) * GPW + col, 16)]
                        for iv in (w & jnp.int32(0xFFFF),
                                   lax.shift_right_logical(w, 16)):
                            ws.append(plsc.load_gather(x0_v, [iv]))
                            ws.append(plsc.load_gather(x1_v, [iv]))
                    sa1 = [[None, None], [None, None]]
                    for src in range(2):
                        aa = [plsc.bitcast(ws[2 * l + src], jnp.bfloat16)
                              for l in range(LL)]
                        for pair in range(2):
                            x1, x2 = aa[2 * pair], aa[2 * pair + 1]
                            m = plsc.bitcast(jnp.minimum(x1, x2), jnp.int32)
                            dw = plsc.bitcast(jnp.abs(x1 - x2), jnp.int32)
                            gh = plsc.load_gather(
                                tab_v, [lax.shift_right_logical(dw, 16)])
                            gl = plsc.load_gather(
                                tab_v, [dw & jnp.int32(0xFFFF)])
                            sa1[src][pair] = (_hi(m) - gh, _lo(m) - gl)
                    for k in range(4):
                        off = k >> 1
                        half = k & 1
                        p = _sa(tab_v, sa1[off][0][half], sa1[off][1][half])
                        cnt = s + 1
                        while cnt % 2 == 0:
                            p = _so(tab_v, stacks[k].pop(), p)
                            cnt //= 2
                        stacks[k].append(p)
                for k in range(4):
                    orefs[k][pl.ds(col, 16)] = stacks[k][0] * GAMMA
                return inner

            lax.fori_loop(0, NGB, gb_body, 0)
            return [pltpu.async_copy(
                        orefs[k],
                        out_hbm.at[pl.ds(brow * GPAD + wid * GPW, GPW)],
                        sem_o)
                    for k, brow in enumerate((2 * ch, 2 * ch + 16,
                                              2 * ch + 1, 2 * ch + 17))]

        def dc_body(i, carry):
            ch_a = 2 * i
            ch_b = 2 * i + 1
            for cp in xcopy(ch_a, xa0_v, xa1_v, sem_a):
                cp.wait()
            for cp in xcopy(ch_b, xb0_v, xb1_v, sem_b):
                cp.start()
            cps_a = compute_chunk(ch_a, xa0_v, xa1_v, orows[:4])
            for cp in xcopy(ch_b, xb0_v, xb1_v, sem_b):
                cp.wait()

            @pl.when(i < (NCH // 2) - 1)
            def _():
                for cp in xcopy(ch_a + 2, xa0_v, xa1_v, sem_a):
                    cp.start()

            cps_b = compute_chunk(ch_b, xb0_v, xb1_v, orows[4:])
            for cp in cps_a + cps_b:
                cp.wait()
            return carry

        lax.fori_loop(0, NCH // 2, dc_body, 0)

    return sc_clause


_SC_CALL_CACHE = []


def kernel(x, I_i):
    if not _SC_CALL_CACHE:
        _SC_CALL_CACHE.append(_make_sc_call())
    sc_clause = _SC_CALL_CACHE[0]
    y16 = (x * INV_GAMMA).astype(jnp.bfloat16)
    hi = lax.bitcast_convert_type(y16[:16], jnp.uint16).astype(jnp.uint32) << 16
    lo = lax.bitcast_convert_type(y16[16:], jnp.uint16).astype(jnp.uint32)
    xp = lax.bitcast_convert_type(hi | lo, jnp.int32)
    idx = I_i.reshape(GG, SL).astype(jnp.int32)
    idx = jnp.pad(idx, ((0, GPAD - GG), (0, 0)))
    idx = idx.reshape(NW, GPW, SL).transpose(0, 2, 1)
    idx = idx[:, 0::2, :] | (idx[:, 1::2, :] << 16)
    out = sc_clause(xp.reshape(-1), idx.reshape(-1), jnp.asarray(_TAB))
    return out.reshape(BB, GPAD)[:, :GG]
